# Initial kernel scaffold; baseline (speedup 1.0000x reference)
import jax
import jax.numpy as jnp
from jax.experimental import pallas as pl


def kernel(adj, x, W1, b1, W2, b2, Wlin, blin):
    raise NotImplementedError("write your pallas kernel here")


# speedup vs baseline: 1.0024x; 1.0024x over previous
import jax
import jax.numpy as jnp
from jax.experimental import pallas as pl
from jax.experimental.pallas import tpu as pltpu


def _prep_kernel(x_ref, W1_ref, W2_ref, b2_ref, Wlin_ref, blin_ref,
                 s1_ref, wv_ref, c_ref):
    s1_ref[...] = jnp.dot(x_ref[...], W1_ref[...],
                          preferred_element_type=jnp.float32)
    wv_ref[...] = jnp.dot(W2_ref[...], Wlin_ref[...],
                          preferred_element_type=jnp.float32)
    c_ref[...] = jnp.dot(b2_ref[...], Wlin_ref[...],
                         preferred_element_type=jnp.float32) + blin_ref[...]


def _layer1_kernel(adj_ref, s1_ref, b1_ref, wv_ref, v_ref):
    h = jnp.dot(adj_ref[...], s1_ref[...],
                preferred_element_type=jnp.float32)
    hr = jnp.maximum(h + b1_ref[...], 0.0)
    v_ref[...] = jnp.dot(hr, wv_ref[...],
                         preferred_element_type=jnp.float32)


def _layer2_kernel(adj_ref, v_ref, c_ref, out_ref):
    out_ref[...] = jnp.dot(adj_ref[...], v_ref[...],
                           preferred_element_type=jnp.float32) + c_ref[...]


def _pick_bm(n):
    for bm in (400, 200, 80, 40, 8):
        if n % bm == 0:
            return bm
    return n


def kernel(adj, x, W1, b1, W2, b2, Wlin, blin):
    n, nf = x.shape
    nh = W1.shape[1]
    bm = _pick_bm(n)

    s1, wv, c = pl.pallas_call(
        _prep_kernel,
        out_shape=[
            jax.ShapeDtypeStruct((n, nh), jnp.float32),
            jax.ShapeDtypeStruct((nh, 1), jnp.float32),
            jax.ShapeDtypeStruct((1, 1), jnp.float32),
        ],
    )(x, W1, W2, b2.reshape(1, nh), Wlin, blin.reshape(1, 1))

    v = pl.pallas_call(
        _layer1_kernel,
        grid=(n // bm,),
        in_specs=[
            pl.BlockSpec((bm, n), lambda i: (i, 0)),
            pl.BlockSpec((n, nh), lambda i: (0, 0)),
            pl.BlockSpec((1, nh), lambda i: (0, 0)),
            pl.BlockSpec((nh, 1), lambda i: (0, 0)),
        ],
        out_specs=pl.BlockSpec((bm, 1), lambda i: (i, 0)),
        out_shape=jax.ShapeDtypeStruct((n, 1), jnp.float32),
        compiler_params=pltpu.CompilerParams(
            dimension_semantics=("parallel",)),
    )(adj, s1, b1.reshape(1, nh), wv)

    out = pl.pallas_call(
        _layer2_kernel,
        grid=(n // bm,),
        in_specs=[
            pl.BlockSpec((bm, n), lambda i: (i, 0)),
            pl.BlockSpec((n, 1), lambda i: (0, 0)),
            pl.BlockSpec((1, 1), lambda i: (0, 0)),
        ],
        out_specs=pl.BlockSpec((bm, 1), lambda i: (i, 0)),
        out_shape=jax.ShapeDtypeStruct((n, 1), jnp.float32),
        compiler_params=pltpu.CompilerParams(
            dimension_semantics=("parallel",)),
    )(adj, v, c)

    return out
